# baseline (speedup 1.0000x reference)
import jax
import jax.numpy as jnp
from jax import lax
from jax.experimental import pallas as pl
from jax.experimental.pallas import tpu as pltpu
from jax.experimental.pallas import tpu_sc as plsc

N_NODES = 100000
N_EDGES = 3200000

LANE = 128
NPAD = 100096
SEG = NPAD // 16
ROWS = 25088
ROWS_PER_TILE = ROWS // 32
CHUNK = 16
NCHUNK = ROWS_PER_TILE // CHUNK

_f32 = jnp.float32
_i32 = jnp.int32

_MESH = plsc.VectorSubcoreMesh(core_axis_name="c", subcore_axis_name="s",
                               num_cores=2, num_subcores=16)


def _sc_pass_body(with_cnt, src_hbm, dst_hbm, tab_hbm, zseg_hbm, drain_hbm,
                  ones_hbm, agg_out, cnt_out, agg_sp, cnt_sp,
                  srcv, dstv, vals, onesv, stage, semg, sems):
    cid = lax.axis_index("c")
    sid = lax.axis_index("s")
    seg = sid * SEG

    pltpu.sync_copy(zseg_hbm, stage)
    pltpu.sync_copy(stage, agg_sp.at[pl.ds(seg, SEG)])
    if with_cnt:
        pltpu.sync_copy(stage, cnt_sp.at[pl.ds(seg, SEG)])
        pltpu.sync_copy(ones_hbm, onesv)
    plsc.subcore_barrier()

    row0 = (cid * 16 + sid) * ROWS_PER_TILE

    def chunk_body(i, carry):
        r = row0 + i * CHUNK
        pltpu.sync_copy(src_hbm.at[pl.ds(r, CHUNK)], srcv)
        pltpu.sync_copy(dst_hbm.at[pl.ds(r, CHUNK)], dstv)

        def fire_gather(j, c):
            pltpu.async_copy(tab_hbm.at[srcv.at[j]], vals.at[j], semg)
            return c

        lax.fori_loop(0, CHUNK, fire_gather, 0)

        if with_cnt:
            def fire_cnt(j, c):
                pltpu.async_copy(onesv, cnt_sp.at[dstv.at[j]], sems, add=True)
                return c

            lax.fori_loop(0, CHUNK, fire_cnt, 0)

        pltpu.make_async_copy(drain_hbm, vals, semg).wait()

        def fire_scatter(j, c):
            pltpu.async_copy(vals.at[j], agg_sp.at[dstv.at[j]], sems, add=True)
            return c

        lax.fori_loop(0, CHUNK, fire_scatter, 0)
        pltpu.make_async_copy(drain_hbm, vals, sems).wait()
        if with_cnt:
            pltpu.make_async_copy(drain_hbm, vals, sems).wait()
        return carry

    lax.fori_loop(0, NCHUNK, chunk_body, 0)
    plsc.subcore_barrier()

    oseg = cid * NPAD + seg
    pltpu.sync_copy(agg_sp.at[pl.ds(seg, SEG)], stage)
    pltpu.sync_copy(stage, agg_out.at[pl.ds(oseg, SEG)])
    if with_cnt:
        pltpu.sync_copy(cnt_sp.at[pl.ds(seg, SEG)], stage)
        pltpu.sync_copy(stage, cnt_out.at[pl.ds(oseg, SEG)])


def _make_sc_pass(with_cnt):
    out_type = [jax.ShapeDtypeStruct((2 * NPAD,), _f32)]
    scratch = [
        pltpu.VMEM_SHARED((NPAD,), _f32),
        pltpu.VMEM((CHUNK, LANE), _i32),
        pltpu.VMEM((CHUNK, LANE), _i32),
        pltpu.VMEM((CHUNK, LANE), _f32),
        pltpu.VMEM((LANE,), _f32),
        pltpu.VMEM((SEG,), _f32),
        pltpu.SemaphoreType.DMA,
        pltpu.SemaphoreType.DMA,
    ]
    if with_cnt:
        out_type = out_type + [jax.ShapeDtypeStruct((2 * NPAD,), _f32)]
        scratch = scratch[:1] + [pltpu.VMEM_SHARED((NPAD,), _f32)] + scratch[1:]

    if with_cnt:
        def body(src_hbm, dst_hbm, tab_hbm, zseg_hbm, drain_hbm, ones_hbm,
                 agg_out, cnt_out, agg_sp, cnt_sp,
                 srcv, dstv, vals, onesv, stage, semg, sems):
            _sc_pass_body(True, src_hbm, dst_hbm, tab_hbm, zseg_hbm, drain_hbm,
                          ones_hbm, agg_out, cnt_out, agg_sp, cnt_sp,
                          srcv, dstv, vals, onesv, stage, semg, sems)
    else:
        def body(src_hbm, dst_hbm, tab_hbm, zseg_hbm, drain_hbm, ones_hbm,
                 agg_out, agg_sp,
                 srcv, dstv, vals, onesv, stage, semg, sems):
            _sc_pass_body(False, src_hbm, dst_hbm, tab_hbm, zseg_hbm, drain_hbm,
                          ones_hbm, agg_out, None, agg_sp, None,
                          srcv, dstv, vals, onesv, stage, semg, sems)

    return pl.kernel(body, out_type=out_type, mesh=_MESH, scratch_types=scratch,
                     name="sage_sc_pass1" if with_cnt else "sage_sc_pass2")


_sc_pass1 = _make_sc_pass(True)
_sc_pass2 = _make_sc_pass(False)


def _mid_body(aggp_ref, cntp_ref, xp_ref, w_ref, s_ref, tpb_ref, degc_ref):
    agg = aggp_ref[0] + aggp_ref[1]
    deg = cntp_ref[0] + cntp_ref[1]
    degc = jnp.maximum(deg, 1.0)
    a = agg / degc
    xv = xp_ref[...]
    s = jnp.zeros_like(a)
    t = jnp.zeros_like(a)
    for k in range(16):
        h = jnp.maximum(a * w_ref[0, k] + xv * w_ref[1, k] + w_ref[2, k], 0.0)
        s = s + w_ref[3, k] * h
        t = t + w_ref[4, k] * h
    s_ref[...] = s
    tpb_ref[...] = t + w_ref[5, 0]
    degc_ref[...] = degc


_mid_tc = pl.pallas_call(
    _mid_body,
    out_shape=[jax.ShapeDtypeStruct((NPAD // LANE, LANE), _f32)] * 3,
    in_specs=[
        pl.BlockSpec(memory_space=pltpu.VMEM),
        pl.BlockSpec(memory_space=pltpu.VMEM),
        pl.BlockSpec(memory_space=pltpu.VMEM),
        pl.BlockSpec(memory_space=pltpu.SMEM),
    ],
    out_specs=[pl.BlockSpec(memory_space=pltpu.VMEM)] * 3,
    name="sage_tc_mid",
)


def _final_body(aggp_ref, degc_ref, tpb_ref, out_ref):
    out_ref[...] = (aggp_ref[0] + aggp_ref[1]) / degc_ref[...] + tpb_ref[...]


_final_tc = pl.pallas_call(
    _final_body,
    out_shape=jax.ShapeDtypeStruct((NPAD // LANE, LANE), _f32),
    in_specs=[pl.BlockSpec(memory_space=pltpu.VMEM)] * 3,
    out_specs=pl.BlockSpec(memory_space=pltpu.VMEM),
    name="sage_tc_final",
)


def kernel(x, edge_index, W1_l, b1, W1_r, W2_l, b2, W2_r):
    xf = x[:, 0].astype(_f32)
    xpad = jnp.concatenate([xf, jnp.zeros((NPAD - N_NODES,), _f32)])

    src = edge_index[0].astype(_i32)
    dst = edge_index[1].astype(_i32)
    npe = ROWS * LANE - N_EDGES
    pad_ids = lax.iota(_i32, npe)
    src_pad = pad_ids % N_NODES
    dst_pad = N_NODES + pad_ids % (NPAD - N_NODES)
    src2d = jnp.concatenate([src, src_pad]).reshape(ROWS, LANE)
    dst2d = jnp.concatenate([dst, dst_pad]).reshape(ROWS, LANE)

    zseg = jnp.zeros((SEG,), _f32)
    drain = jnp.zeros((CHUNK, LANE), _f32)
    ones = jnp.ones((LANE,), _f32)
    w = jnp.stack([
        W1_l[:, 0], W1_r[:, 0], b1, W2_l[0, :], W2_r[0, :],
        jnp.full((16,), b2[0], dtype=_f32),
    ]).astype(_f32)

    agg1p, cntp = _sc_pass1(src2d, dst2d, xpad, zseg, drain, ones)
    s, tpb, degc = _mid_tc(
        agg1p.reshape(2, NPAD // LANE, LANE),
        cntp.reshape(2, NPAD // LANE, LANE),
        xpad.reshape(NPAD // LANE, LANE), w)
    (agg2p,) = _sc_pass2(src2d, dst2d, s.reshape(NPAD), zseg, drain, ones)
    out = _final_tc(agg2p.reshape(2, NPAD // LANE, LANE), degc, tpb)
    return out.reshape(NPAD)[:N_NODES].reshape(N_NODES, 1)


# speedup vs baseline: 1.4980x; 1.4980x over previous
import jax
import jax.numpy as jnp
from jax import lax
from jax.experimental import pallas as pl
from jax.experimental.pallas import tpu as pltpu
from jax.experimental.pallas import tpu_sc as plsc

N_NODES = 100000
N_EDGES = 3200000

LANE = 128
NPAD = 100096
SEG = NPAD // 16
ROWS = 25088
ROWS_PER_TILE = ROWS // 32
CHUNK = 16
NCHUNK = ROWS_PER_TILE // CHUNK

_f32 = jnp.float32
_i32 = jnp.int32

_MESH = plsc.VectorSubcoreMesh(core_axis_name="c", subcore_axis_name="s",
                               num_cores=2, num_subcores=16)


def _sc_pass_body(with_cnt, src_hbm, dst_hbm, tab_hbm, zseg_hbm, drain_hbm,
                  ones_hbm, agg_out, cnt_out, tab_sp, agg_sp, cnt_sp,
                  srcv, dstv, vals, onesv, stage, semg, sems):
    cid = lax.axis_index("c")
    sid = lax.axis_index("s")
    seg = sid * SEG

    pltpu.sync_copy(zseg_hbm, stage)
    pltpu.sync_copy(stage, agg_sp.at[pl.ds(seg, SEG)])
    if with_cnt:
        pltpu.sync_copy(stage, cnt_sp.at[pl.ds(seg, SEG)])
        pltpu.sync_copy(ones_hbm, onesv)
    pltpu.sync_copy(tab_hbm.at[pl.ds(seg, SEG)], stage)
    pltpu.sync_copy(stage, tab_sp.at[pl.ds(seg, SEG)])
    plsc.subcore_barrier()

    row0 = (cid * 16 + sid) * ROWS_PER_TILE

    def chunk_body(i, carry):
        r = row0 + i * CHUNK
        pltpu.sync_copy(src_hbm.at[pl.ds(r, CHUNK)], srcv)
        pltpu.sync_copy(dst_hbm.at[pl.ds(r, CHUNK)], dstv)

        def fire_gather(j, c):
            pltpu.async_copy(tab_sp.at[srcv.at[j]], vals.at[j], semg)
            return c

        lax.fori_loop(0, CHUNK, fire_gather, 0)

        if with_cnt:
            def fire_cnt(j, c):
                pltpu.async_copy(onesv, cnt_sp.at[dstv.at[j]], sems, add=True)
                return c

            lax.fori_loop(0, CHUNK, fire_cnt, 0)

        pltpu.make_async_copy(drain_hbm, vals, semg).wait()

        def fire_scatter(j, c):
            pltpu.async_copy(vals.at[j], agg_sp.at[dstv.at[j]], sems, add=True)
            return c

        lax.fori_loop(0, CHUNK, fire_scatter, 0)
        pltpu.make_async_copy(drain_hbm, vals, sems).wait()
        if with_cnt:
            pltpu.make_async_copy(drain_hbm, vals, sems).wait()
        return carry

    lax.fori_loop(0, NCHUNK, chunk_body, 0)
    plsc.subcore_barrier()

    oseg = cid * NPAD + seg
    pltpu.sync_copy(agg_sp.at[pl.ds(seg, SEG)], stage)
    pltpu.sync_copy(stage, agg_out.at[pl.ds(oseg, SEG)])
    if with_cnt:
        pltpu.sync_copy(cnt_sp.at[pl.ds(seg, SEG)], stage)
        pltpu.sync_copy(stage, cnt_out.at[pl.ds(oseg, SEG)])


def _make_sc_pass(with_cnt):
    out_type = [jax.ShapeDtypeStruct((2 * NPAD,), _f32)]
    scratch = [
        pltpu.VMEM_SHARED((NPAD,), _f32),
        pltpu.VMEM_SHARED((NPAD,), _f32),
        pltpu.VMEM((CHUNK, LANE), _i32),
        pltpu.VMEM((CHUNK, LANE), _i32),
        pltpu.VMEM((CHUNK, LANE), _f32),
        pltpu.VMEM((LANE,), _f32),
        pltpu.VMEM((SEG,), _f32),
        pltpu.SemaphoreType.DMA,
        pltpu.SemaphoreType.DMA,
    ]
    if with_cnt:
        out_type = out_type + [jax.ShapeDtypeStruct((2 * NPAD,), _f32)]
        scratch = scratch[:2] + [pltpu.VMEM_SHARED((NPAD,), _f32)] + scratch[2:]

    if with_cnt:
        def body(src_hbm, dst_hbm, tab_hbm, zseg_hbm, drain_hbm, ones_hbm,
                 agg_out, cnt_out, tab_sp, agg_sp, cnt_sp,
                 srcv, dstv, vals, onesv, stage, semg, sems):
            _sc_pass_body(True, src_hbm, dst_hbm, tab_hbm, zseg_hbm, drain_hbm,
                          ones_hbm, agg_out, cnt_out, tab_sp, agg_sp, cnt_sp,
                          srcv, dstv, vals, onesv, stage, semg, sems)
    else:
        def body(src_hbm, dst_hbm, tab_hbm, zseg_hbm, drain_hbm, ones_hbm,
                 agg_out, tab_sp, agg_sp,
                 srcv, dstv, vals, onesv, stage, semg, sems):
            _sc_pass_body(False, src_hbm, dst_hbm, tab_hbm, zseg_hbm, drain_hbm,
                          ones_hbm, agg_out, None, tab_sp, agg_sp, None,
                          srcv, dstv, vals, onesv, stage, semg, sems)

    return pl.kernel(body, out_type=out_type, mesh=_MESH, scratch_types=scratch,
                     name="sage_sc_pass1" if with_cnt else "sage_sc_pass2")


_sc_pass1 = _make_sc_pass(True)
_sc_pass2 = _make_sc_pass(False)


def _mid_body(aggp_ref, cntp_ref, xp_ref, w_ref, s_ref, tpb_ref, degc_ref):
    agg = aggp_ref[0] + aggp_ref[1]
    deg = cntp_ref[0] + cntp_ref[1]
    degc = jnp.maximum(deg, 1.0)
    a = agg / degc
    xv = xp_ref[...]
    s = jnp.zeros_like(a)
    t = jnp.zeros_like(a)
    for k in range(16):
        h = jnp.maximum(a * w_ref[0, k] + xv * w_ref[1, k] + w_ref[2, k], 0.0)
        s = s + w_ref[3, k] * h
        t = t + w_ref[4, k] * h
    s_ref[...] = s
    tpb_ref[...] = t + w_ref[5, 0]
    degc_ref[...] = degc


_mid_tc = pl.pallas_call(
    _mid_body,
    out_shape=[jax.ShapeDtypeStruct((NPAD // LANE, LANE), _f32)] * 3,
    in_specs=[
        pl.BlockSpec(memory_space=pltpu.VMEM),
        pl.BlockSpec(memory_space=pltpu.VMEM),
        pl.BlockSpec(memory_space=pltpu.VMEM),
        pl.BlockSpec(memory_space=pltpu.SMEM),
    ],
    out_specs=[pl.BlockSpec(memory_space=pltpu.VMEM)] * 3,
    name="sage_tc_mid",
)


def _final_body(aggp_ref, degc_ref, tpb_ref, out_ref):
    out_ref[...] = (aggp_ref[0] + aggp_ref[1]) / degc_ref[...] + tpb_ref[...]


_final_tc = pl.pallas_call(
    _final_body,
    out_shape=jax.ShapeDtypeStruct((NPAD // LANE, LANE), _f32),
    in_specs=[pl.BlockSpec(memory_space=pltpu.VMEM)] * 3,
    out_specs=pl.BlockSpec(memory_space=pltpu.VMEM),
    name="sage_tc_final",
)


def kernel(x, edge_index, W1_l, b1, W1_r, W2_l, b2, W2_r):
    xf = x[:, 0].astype(_f32)
    xpad = jnp.concatenate([xf, jnp.zeros((NPAD - N_NODES,), _f32)])

    src = edge_index[0].astype(_i32)
    dst = edge_index[1].astype(_i32)
    npe = ROWS * LANE - N_EDGES
    pad_ids = lax.iota(_i32, npe)
    src_pad = pad_ids % N_NODES
    dst_pad = N_NODES + pad_ids % (NPAD - N_NODES)
    src2d = jnp.concatenate([src, src_pad]).reshape(ROWS, LANE)
    dst2d = jnp.concatenate([dst, dst_pad]).reshape(ROWS, LANE)

    zseg = jnp.zeros((SEG,), _f32)
    drain = jnp.zeros((CHUNK, LANE), _f32)
    ones = jnp.ones((LANE,), _f32)
    w = jnp.stack([
        W1_l[:, 0], W1_r[:, 0], b1, W2_l[0, :], W2_r[0, :],
        jnp.full((16,), b2[0], dtype=_f32),
    ]).astype(_f32)

    agg1p, cntp = _sc_pass1(src2d, dst2d, xpad, zseg, drain, ones)
    s, tpb, degc = _mid_tc(
        agg1p.reshape(2, NPAD // LANE, LANE),
        cntp.reshape(2, NPAD // LANE, LANE),
        xpad.reshape(NPAD // LANE, LANE), w)
    (agg2p,) = _sc_pass2(src2d, dst2d, s.reshape(NPAD), zseg, drain, ones)
    out = _final_tc(agg2p.reshape(2, NPAD // LANE, LANE), degc, tpb)
    return out.reshape(NPAD)[:N_NODES].reshape(N_NODES, 1)
